# Initial kernel scaffold; baseline (speedup 1.0000x reference)
import jax
import jax.numpy as jnp
from jax.experimental import pallas as pl


def kernel(nei_h_0, nei_h_1, nei_h_2, rel_0, rel_1, rel_2, target_features, att0, att2_0, Wr0, br0, Wr2_0, br2_0, aggW0, aggb0, aggatt0, att1, att2_1, Wr1, br1, Wr2_1, br2_1, aggW1, aggb1, aggatt1, interW, interb, interatt, sele_nei_0, sele_nei_1):
    raise NotImplementedError("write your pallas kernel here")


# speedup vs baseline: 3.0168x; 3.0168x over previous
import functools

import jax
import jax.numpy as jnp
from jax import lax
from jax.experimental import pallas as pl
from jax.experimental.pallas import tpu as pltpu
from jax.experimental.pallas import tpu_sc as plsc

D = 128
N = 50000
M = 50000
K = 10
LANES = 16
BLK = 2000
GRID = N // BLK
NCHUNK = N // LANES
NWORK = 32
CPW = -(-NCHUNK // NWORK)


def _elu(x):
    return jnp.where(x > 0, x, jnp.exp(x) - 1.0)


def _phase_a_body(nh0, r0, nh1, rl1, nh2, rl2,
                  Wr0, Wr20, Wr1, Wr21,
                  att0, att20, att1, att21,
                  br0, br20, br1, br21, out_n, out_m):
    cfgs = (
        (nh0, Wr0, br0, att0, nh1),
        (r0, Wr20, br20, att20, rl1),
        (nh0, Wr1, br1, att1, nh2),
        (r0, Wr21, br21, att21, rl2),
    )
    s1s, l1s, ps, qs = [], [], [], []
    for ref, W, bias, att, tab in cfgs:
        hr = jnp.dot(ref[...], W[...].T, preferred_element_type=jnp.float32)
        hr = hr + bias[...][None, :]
        a1 = att[0, :D]
        a2 = att[0, D:]
        s1s.append(jnp.dot(hr, a1, preferred_element_type=jnp.float32))
        l1s.append(jnp.sum(jnp.abs(hr), axis=1))
        t = tab[...]
        ps.append(jnp.dot(t, a2, preferred_element_type=jnp.float32))
        qs.append(jnp.sum(jnp.abs(t), axis=1))
    out_n[...] = jnp.stack(s1s + l1s, axis=1)
    out_m[...] = jnp.stack(ps + qs, axis=1)


def _run_phase_a(nh0, r0, nh1, rl1, nh2, rl2, Ws, atts, brs):
    blk = pl.BlockSpec((BLK, D), lambda i: (i, 0))
    full = lambda a: pl.BlockSpec(a.shape, lambda i: tuple(0 for _ in a.shape))
    ins = [nh0, r0, nh1, rl1, nh2, rl2] + list(Ws) + list(atts) + list(brs)
    in_specs = [blk] * 6 + [full(a) for a in ins[6:]]
    return pl.pallas_call(
        _phase_a_body,
        grid=(GRID,),
        in_specs=in_specs,
        out_specs=[pl.BlockSpec((BLK, 8), lambda i: (i, 0))] * 2,
        out_shape=[jax.ShapeDtypeStruct((N, 8), jnp.float32),
                   jax.ShapeDtypeStruct((M, 8), jnp.float32)],
    )(*ins)


def _gather_body(table, idxF, s1h, l1h, ph, qh, out,
                 p_v, q_v, idx_v, s1_v, l1_v, w_v, row_v, acc_v,
                 sem0, sem1):
    info = plsc.get_sparse_core_info()
    nc = info.num_cores
    wid = lax.axis_index("s") * nc + lax.axis_index("c")
    pltpu.sync_copy(ph, p_v)
    pltpu.sync_copy(qh, q_v)
    start = wid * CPW
    end = jnp.minimum(start + CPW, NCHUNK)

    def chunk(c, carry):
        base = c * LANES
        pltpu.sync_copy(idxF.at[pl.ds(c * (K * LANES), K * LANES)], idx_v)
        pltpu.sync_copy(s1h.at[pl.ds(base, LANES)], s1_v)
        pltpu.sync_copy(l1h.at[pl.ds(base, LANES)], l1_v)
        s1r = s1_v[...]
        l1r = l1_v[...]
        es = []
        for k in range(K):
            ii = idx_v[pl.ds(k * LANES, LANES)]
            pk = plsc.load_gather(p_v, [ii])
            qk = plsc.load_gather(q_v, [ii])
            e = (s1r + pk) / jnp.maximum(l1r + qk, 1e-12)
            es.append(jnp.where(e > 0, e, 0.01 * e))
        m = es[0]
        for k in range(1, K):
            m = jnp.maximum(m, es[k])
        exps = [jnp.exp(e - m) for e in es]
        ssum = exps[0]
        for k in range(1, K):
            ssum = ssum + exps[k]
        inv = 1.0 / ssum
        for k in range(K):
            w_v[k] = exps[k] * inv

        def fire(k):
            sem = sem0 if k % 2 == 0 else sem1
            pltpu.make_async_copy(
                table.at[idx_v.at[pl.ds(k * LANES, LANES)]],
                row_v.at[k % 2], sem).start()

        fire(0)
        for k in range(K):
            if k + 1 < K:
                fire(k + 1)
            sem = sem0 if k % 2 == 0 else sem1
            pltpu.make_async_copy(
                table.at[idx_v.at[pl.ds(k * LANES, LANES)]],
                row_v.at[k % 2], sem).wait()
            buf = k % 2

            def accum(n, cc):
                kf = jnp.full((LANES,), k, jnp.int32)
                lane = jnp.full((LANES,), n, jnp.int32)
                w = plsc.load_gather(w_v, [kf, lane])
                for j in range(D // LANES):
                    sl = pl.ds(j * LANES, LANES)
                    r = row_v[buf, n, sl]
                    if k == 0:
                        acc_v[n, sl] = w * r
                    else:
                        acc_v[n, sl] = acc_v[n, sl] + w * r
                return cc

            lax.fori_loop(0, LANES, accum, 0)
        pltpu.sync_copy(acc_v, out.at[pl.ds(base, LANES), :])
        return carry

    lax.fori_loop(start, end, chunk, 0)


@functools.lru_cache(maxsize=None)
def _get_gather_call():
    mesh = plsc.VectorSubcoreMesh(core_axis_name="c", subcore_axis_name="s")
    return pl.kernel(
        _gather_body,
        out_type=jax.ShapeDtypeStruct((N, D), jnp.float32),
        mesh=mesh,
        compiler_params=pltpu.CompilerParams(needs_layout_passes=False),
        scratch_types=[
        pltpu.VMEM((M,), jnp.float32),
        pltpu.VMEM((M,), jnp.float32),
        pltpu.VMEM((K * LANES,), jnp.int32),
        pltpu.VMEM((LANES,), jnp.float32),
        pltpu.VMEM((LANES,), jnp.float32),
        pltpu.VMEM((K, LANES), jnp.float32),
        pltpu.VMEM((2, LANES, D), jnp.float32),
        pltpu.VMEM((LANES, D), jnp.float32),
        pltpu.SemaphoreType.DMA,
        pltpu.SemaphoreType.DMA,
        ],
    )


def _phase_c_body(a0, r0, a1, r1, w0, b0, w1, b1, out):
    i = pl.program_id(0)
    rows = []
    for em, W, bb in ((a0, w0, b0), (r0, w0, b0), (a1, w1, b1), (r1, w1, b1)):
        t = jnp.tanh(jnp.dot(em[...], W[...].T,
                             preferred_element_type=jnp.float32)
                     + bb[...][None, :])
        rows.append(jnp.sum(t, axis=0))
    val = jnp.stack(rows, axis=0)

    @pl.when(i == 0)
    def _():
        out[...] = val

    @pl.when(i > 0)
    def _():
        out[...] = out[...] + val


def _run_phase_c(a0, r0, a1, r1, w0, b0, w1, b1):
    blk = pl.BlockSpec((BLK, D), lambda i: (i, 0))
    full = lambda a: pl.BlockSpec(a.shape, lambda i: tuple(0 for _ in a.shape))
    return pl.pallas_call(
        _phase_c_body,
        grid=(GRID,),
        in_specs=[blk] * 4 + [full(w0), full(b0), full(w1), full(b1)],
        out_specs=pl.BlockSpec((4, D), lambda i: (0, 0)),
        out_shape=jax.ShapeDtypeStruct((4, D), jnp.float32),
    )(a0, r0, a1, r1, w0, b0, w1, b1)


def _phase_d_body(a0, r0, a1, r1, tf, W, bb, bet, out):
    i = pl.program_id(0)
    ba0 = bet[0, 0]
    br0 = bet[0, 1]
    ba1 = bet[0, 2]
    br1 = bet[0, 3]
    e0 = _elu(ba0 * a0[...] + br0 * r0[...])
    e1 = _elu(ba1 * a1[...] + br1 * r1[...])
    rows = []
    for em in (e0, e1, tf[...]):
        t = jnp.tanh(jnp.dot(em, W[...].T,
                             preferred_element_type=jnp.float32)
                     + bb[...][None, :])
        rows.append(jnp.sum(t, axis=0))
    rows.append(jnp.zeros((D,), jnp.float32))
    val = jnp.stack(rows, axis=0)

    @pl.when(i == 0)
    def _():
        out[...] = val

    @pl.when(i > 0)
    def _():
        out[...] = out[...] + val


def _run_phase_d(a0, r0, a1, r1, tf, W, bb, bet):
    blk = pl.BlockSpec((BLK, D), lambda i: (i, 0))
    full = lambda a: pl.BlockSpec(a.shape, lambda i: tuple(0 for _ in a.shape))
    return pl.pallas_call(
        _phase_d_body,
        grid=(GRID,),
        in_specs=[blk] * 5 + [full(W), full(bb), full(bet)],
        out_specs=pl.BlockSpec((4, D), lambda i: (0, 0)),
        out_shape=jax.ShapeDtypeStruct((4, D), jnp.float32),
    )(a0, r0, a1, r1, tf, W, bb, bet)


def _phase_e_body(a0, r0, a1, r1, tf, bet, out):
    ba0 = bet[0, 0]
    br0 = bet[0, 1]
    ba1 = bet[0, 2]
    br1 = bet[0, 3]
    ib0 = bet[0, 4]
    ib1 = bet[0, 5]
    ib2 = bet[0, 6]
    e0 = _elu(ba0 * a0[...] + br0 * r0[...])
    e1 = _elu(ba1 * a1[...] + br1 * r1[...])
    out[...] = ib0 * e0 + ib1 * e1 + ib2 * tf[...]


def _run_phase_e(a0, r0, a1, r1, tf, bet):
    blk = pl.BlockSpec((BLK, D), lambda i: (i, 0))
    full = lambda a: pl.BlockSpec(a.shape, lambda i: tuple(0 for _ in a.shape))
    return pl.pallas_call(
        _phase_e_body,
        grid=(GRID,),
        in_specs=[blk] * 5 + [full(bet)],
        out_specs=blk,
        out_shape=jax.ShapeDtypeStruct((N, D), jnp.float32),
    )(a0, r0, a1, r1, tf, bet)


def kernel(nei_h_0, nei_h_1, nei_h_2, rel_0, rel_1, rel_2, target_features,
           att0, att2_0, Wr0, br0, Wr2_0, br2_0, aggW0, aggb0, aggatt0,
           att1, att2_1, Wr1, br1, Wr2_1, br2_1, aggW1, aggb1, aggatt1,
           interW, interb, interatt, sele_nei_0, sele_nei_1):
    f32 = jnp.float32

    def _chunk_kmajor(nei):
        r = nei.reshape(NCHUNK, LANES, K).astype(jnp.int32)
        return jnp.ravel(jnp.transpose(r, (0, 2, 1)))

    idxF0 = _chunk_kmajor(sele_nei_0)
    idxF1 = _chunk_kmajor(sele_nei_1)

    scal_n, scal_m = _run_phase_a(
        nei_h_0, rel_0, nei_h_1, rel_1, nei_h_2, rel_2,
        (Wr0, Wr2_0, Wr1, Wr2_1),
        (att0, att2_0, att1, att2_1),
        (br0, br2_0, br1, br2_1))
    sn = jnp.transpose(scal_n)
    sm = jnp.transpose(scal_m)

    tables = (nei_h_1, rel_1, nei_h_2, rel_2)
    idxTs = (idxF0, idxF0, idxF1, idxF1)
    gather_call = _get_gather_call()
    embs = []
    for b in range(4):
        embs.append(gather_call(
            tables[b], idxTs[b], sn[b], sn[4 + b], sm[b], sm[4 + b]))
    embA0, embR0, embA1, embR1 = embs

    sp = _run_phase_c(embA0, embR0, embA1, embR1,
                      aggW0, aggb0, aggW1, aggb1) / N
    bi0 = jax.nn.softmax(jnp.stack([jnp.dot(aggatt0[0], sp[0]),
                                    jnp.dot(aggatt0[0], sp[1])]))
    bi1 = jax.nn.softmax(jnp.stack([jnp.dot(aggatt1[0], sp[2]),
                                    jnp.dot(aggatt1[0], sp[3])]))
    bet4 = jnp.zeros((1, D), f32).at[0, :4].set(
        jnp.stack([bi0[0], bi0[1], bi1[0], bi1[1]]))

    sp2 = _run_phase_d(embA0, embR0, embA1, embR1, target_features,
                       interW, interb, bet4) / N
    ib = jax.nn.softmax(jnp.stack([jnp.dot(interatt[0], sp2[0]),
                                   jnp.dot(interatt[0], sp2[1]),
                                   jnp.dot(interatt[0], sp2[2])]))
    bet7 = bet4.at[0, 4:7].set(ib)

    return _run_phase_e(embA0, embR0, embA1, embR1, target_features, bet7)


# speedup vs baseline: 3.3759x; 1.1190x over previous
import functools

import jax
import jax.numpy as jnp
from jax import lax
from jax.experimental import pallas as pl
from jax.experimental.pallas import tpu as pltpu
from jax.experimental.pallas import tpu_sc as plsc

D = 128
N = 50000
M = 50000
K = 10
LANES = 16
BLK = 2000
GRID = N // BLK
NCHUNK = N // LANES
NWORK = 32
CPW = -(-NCHUNK // NWORK)
PADC = NWORK * CPW
PADN = PADC * LANES
PADI = PADC * K * LANES
NBUF = 3


def _elu(x):
    return jnp.where(x > 0, x, jnp.exp(x) - 1.0)


def _phase_a_body(nh0, r0, nh1, rl1, nh2, rl2,
                  Wr0, Wr20, Wr1, Wr21,
                  att0, att20, att1, att21,
                  br0, br20, br1, br21, out_n, out_m):
    cfgs = (
        (nh0, Wr0, br0, att0, nh1),
        (r0, Wr20, br20, att20, rl1),
        (nh0, Wr1, br1, att1, nh2),
        (r0, Wr21, br21, att21, rl2),
    )
    s1s, l1s, ps, qs = [], [], [], []
    for ref, W, bias, att, tab in cfgs:
        hr = jnp.dot(ref[...], W[...].T, preferred_element_type=jnp.float32)
        hr = hr + bias[...][None, :]
        a1 = att[0, :D]
        a2 = att[0, D:]
        s1s.append(jnp.dot(hr, a1, preferred_element_type=jnp.float32))
        l1s.append(jnp.sum(jnp.abs(hr), axis=1))
        t = tab[...]
        ps.append(jnp.dot(t, a2, preferred_element_type=jnp.float32))
        qs.append(jnp.sum(jnp.abs(t), axis=1))
    out_n[...] = jnp.stack(s1s + l1s, axis=1)
    out_m[...] = jnp.stack(ps + qs, axis=1)


def _run_phase_a(nh0, r0, nh1, rl1, nh2, rl2, Ws, atts, brs):
    blk = pl.BlockSpec((BLK, D), lambda i: (i, 0))
    full = lambda a: pl.BlockSpec(a.shape, lambda i: tuple(0 for _ in a.shape))
    ins = [nh0, r0, nh1, rl1, nh2, rl2] + list(Ws) + list(atts) + list(brs)
    in_specs = [blk] * 6 + [full(a) for a in ins[6:]]
    return pl.pallas_call(
        _phase_a_body,
        grid=(GRID,),
        in_specs=in_specs,
        out_specs=[pl.BlockSpec((BLK, 8), lambda i: (i, 0))] * 2,
        out_shape=[jax.ShapeDtypeStruct((N, 8), jnp.float32),
                   jax.ShapeDtypeStruct((M, 8), jnp.float32)],
    )(*ins)


def _gather_body(table, idxF, s1h, l1h, ph, qh, out,
                 p_v, q_v, idx_a, s1_a, l1_a, w_v, row_v, acc_v,
                 sa, sb, sc):
    sems = (sa, sb, sc)
    info = plsc.get_sparse_core_info()
    nc = info.num_cores
    wid = lax.axis_index("s") * nc + lax.axis_index("c")
    pltpu.sync_copy(ph, p_v)
    pltpu.sync_copy(qh, q_v)
    start = wid * CPW
    end = jnp.minimum(start + CPW, NCHUNK)
    pltpu.sync_copy(idxF.at[pl.ds(start * (K * LANES), CPW * K * LANES)],
                    idx_a)
    pltpu.sync_copy(s1h.at[pl.ds(start * LANES, CPW * LANES)], s1_a)
    pltpu.sync_copy(l1h.at[pl.ds(start * LANES, CPW * LANES)], l1_a)

    def chunk(c, carry):
        cl = c - start
        ibase = cl * (K * LANES)
        nbase = cl * LANES
        s1r = s1_a[pl.ds(nbase, LANES)]
        l1r = l1_a[pl.ds(nbase, LANES)]
        es = []
        for k in range(K):
            ii = idx_a[pl.ds(ibase + k * LANES, LANES)]
            pk = plsc.load_gather(p_v, [ii])
            qk = plsc.load_gather(q_v, [ii])
            e = (s1r + pk) / jnp.maximum(l1r + qk, 1e-12)
            es.append(jnp.where(e > 0, e, 0.01 * e))
        m = es[0]
        for k in range(1, K):
            m = jnp.maximum(m, es[k])
        exps = [jnp.exp(e - m) for e in es]
        ssum = exps[0]
        for k in range(1, K):
            ssum = ssum + exps[k]
        inv = 1.0 / ssum
        for k in range(K):
            w_v[k] = exps[k] * inv

        def fire(k):
            pltpu.make_async_copy(
                table.at[idx_a.at[pl.ds(ibase + k * LANES, LANES)]],
                row_v.at[k % NBUF], sems[k % NBUF]).start()

        for k in range(NBUF):
            fire(k)
        for k in range(K):
            buf = k % NBUF
            pltpu.make_async_copy(
                table.at[idx_a.at[pl.ds(ibase + k * LANES, LANES)]],
                row_v.at[buf], sems[buf]).wait()

            def accum(n, cc):
                kf = jnp.full((LANES,), k, jnp.int32)
                lane = jnp.full((LANES,), n, jnp.int32)
                w = plsc.load_gather(w_v, [kf, lane])
                for j in range(D // LANES):
                    sl = pl.ds(j * LANES, LANES)
                    r = row_v[buf, n, sl]
                    if k == 0:
                        acc_v[n, sl] = w * r
                    else:
                        acc_v[n, sl] = acc_v[n, sl] + w * r
                return cc

            lax.fori_loop(0, LANES, accum, 0)
            if k + NBUF < K:
                fire(k + NBUF)
        pltpu.sync_copy(acc_v, out.at[pl.ds(c * LANES, LANES), :])
        return carry

    lax.fori_loop(start, end, chunk, 0)


@functools.lru_cache(maxsize=None)
def _get_gather_call():
    mesh = plsc.VectorSubcoreMesh(core_axis_name="c", subcore_axis_name="s")
    return pl.kernel(
        _gather_body,
        out_type=jax.ShapeDtypeStruct((N, D), jnp.float32),
        mesh=mesh,
        compiler_params=pltpu.CompilerParams(needs_layout_passes=False),
        scratch_types=[
        pltpu.VMEM((M,), jnp.float32),
        pltpu.VMEM((M,), jnp.float32),
        pltpu.VMEM((CPW * K * LANES,), jnp.int32),
        pltpu.VMEM((CPW * LANES,), jnp.float32),
        pltpu.VMEM((CPW * LANES,), jnp.float32),
        pltpu.VMEM((K, LANES), jnp.float32),
        pltpu.VMEM((NBUF, LANES, D), jnp.float32),
        pltpu.VMEM((LANES, D), jnp.float32),
        pltpu.SemaphoreType.DMA,
        pltpu.SemaphoreType.DMA,
        pltpu.SemaphoreType.DMA,
        ],
    )


def _phase_c_body(a0, r0, a1, r1, w0, b0, w1, b1, out):
    i = pl.program_id(0)
    rows = []
    for em, W, bb in ((a0, w0, b0), (r0, w0, b0), (a1, w1, b1), (r1, w1, b1)):
        t = jnp.tanh(jnp.dot(em[...], W[...].T,
                             preferred_element_type=jnp.float32)
                     + bb[...][None, :])
        rows.append(jnp.sum(t, axis=0))
    val = jnp.stack(rows, axis=0)

    @pl.when(i == 0)
    def _():
        out[...] = val

    @pl.when(i > 0)
    def _():
        out[...] = out[...] + val


def _run_phase_c(a0, r0, a1, r1, w0, b0, w1, b1):
    blk = pl.BlockSpec((BLK, D), lambda i: (i, 0))
    full = lambda a: pl.BlockSpec(a.shape, lambda i: tuple(0 for _ in a.shape))
    return pl.pallas_call(
        _phase_c_body,
        grid=(GRID,),
        in_specs=[blk] * 4 + [full(w0), full(b0), full(w1), full(b1)],
        out_specs=pl.BlockSpec((4, D), lambda i: (0, 0)),
        out_shape=jax.ShapeDtypeStruct((4, D), jnp.float32),
    )(a0, r0, a1, r1, w0, b0, w1, b1)


def _phase_d_body(a0, r0, a1, r1, tf, W, bb, bet, out):
    i = pl.program_id(0)
    ba0 = bet[0, 0]
    br0 = bet[0, 1]
    ba1 = bet[0, 2]
    br1 = bet[0, 3]
    e0 = _elu(ba0 * a0[...] + br0 * r0[...])
    e1 = _elu(ba1 * a1[...] + br1 * r1[...])
    rows = []
    for em in (e0, e1, tf[...]):
        t = jnp.tanh(jnp.dot(em, W[...].T,
                             preferred_element_type=jnp.float32)
                     + bb[...][None, :])
        rows.append(jnp.sum(t, axis=0))
    rows.append(jnp.zeros((D,), jnp.float32))
    val = jnp.stack(rows, axis=0)

    @pl.when(i == 0)
    def _():
        out[...] = val

    @pl.when(i > 0)
    def _():
        out[...] = out[...] + val


def _run_phase_d(a0, r0, a1, r1, tf, W, bb, bet):
    blk = pl.BlockSpec((BLK, D), lambda i: (i, 0))
    full = lambda a: pl.BlockSpec(a.shape, lambda i: tuple(0 for _ in a.shape))
    return pl.pallas_call(
        _phase_d_body,
        grid=(GRID,),
        in_specs=[blk] * 5 + [full(W), full(bb), full(bet)],
        out_specs=pl.BlockSpec((4, D), lambda i: (0, 0)),
        out_shape=jax.ShapeDtypeStruct((4, D), jnp.float32),
    )(a0, r0, a1, r1, tf, W, bb, bet)


def _phase_e_body(a0, r0, a1, r1, tf, bet, out):
    ba0 = bet[0, 0]
    br0 = bet[0, 1]
    ba1 = bet[0, 2]
    br1 = bet[0, 3]
    ib0 = bet[0, 4]
    ib1 = bet[0, 5]
    ib2 = bet[0, 6]
    e0 = _elu(ba0 * a0[...] + br0 * r0[...])
    e1 = _elu(ba1 * a1[...] + br1 * r1[...])
    out[...] = ib0 * e0 + ib1 * e1 + ib2 * tf[...]


def _run_phase_e(a0, r0, a1, r1, tf, bet):
    blk = pl.BlockSpec((BLK, D), lambda i: (i, 0))
    full = lambda a: pl.BlockSpec(a.shape, lambda i: tuple(0 for _ in a.shape))
    return pl.pallas_call(
        _phase_e_body,
        grid=(GRID,),
        in_specs=[blk] * 5 + [full(bet)],
        out_specs=blk,
        out_shape=jax.ShapeDtypeStruct((N, D), jnp.float32),
    )(a0, r0, a1, r1, tf, bet)


def kernel(nei_h_0, nei_h_1, nei_h_2, rel_0, rel_1, rel_2, target_features,
           att0, att2_0, Wr0, br0, Wr2_0, br2_0, aggW0, aggb0, aggatt0,
           att1, att2_1, Wr1, br1, Wr2_1, br2_1, aggW1, aggb1, aggatt1,
           interW, interb, interatt, sele_nei_0, sele_nei_1):
    f32 = jnp.float32

    def _chunk_kmajor(nei):
        r = nei.reshape(NCHUNK, LANES, K).astype(jnp.int32)
        flat = jnp.ravel(jnp.transpose(r, (0, 2, 1)))
        return jnp.pad(flat, (0, PADI - flat.shape[0]))

    idxF0 = _chunk_kmajor(sele_nei_0)
    idxF1 = _chunk_kmajor(sele_nei_1)

    scal_n, scal_m = _run_phase_a(
        nei_h_0, rel_0, nei_h_1, rel_1, nei_h_2, rel_2,
        (Wr0, Wr2_0, Wr1, Wr2_1),
        (att0, att2_0, att1, att2_1),
        (br0, br2_0, br1, br2_1))
    sn = jnp.pad(jnp.transpose(scal_n),
                 ((0, 0), (0, PADN - N)))
    sm = jnp.transpose(scal_m)

    tables = (nei_h_1, rel_1, nei_h_2, rel_2)
    idxTs = (idxF0, idxF0, idxF1, idxF1)
    gather_call = _get_gather_call()
    embs = []
    for b in range(4):
        embs.append(gather_call(
            tables[b], idxTs[b], sn[b], sn[4 + b], sm[b], sm[4 + b]))
    embA0, embR0, embA1, embR1 = embs

    sp = _run_phase_c(embA0, embR0, embA1, embR1,
                      aggW0, aggb0, aggW1, aggb1) / N
    bi0 = jax.nn.softmax(jnp.stack([jnp.dot(aggatt0[0], sp[0]),
                                    jnp.dot(aggatt0[0], sp[1])]))
    bi1 = jax.nn.softmax(jnp.stack([jnp.dot(aggatt1[0], sp[2]),
                                    jnp.dot(aggatt1[0], sp[3])]))
    bet4 = jnp.zeros((1, D), f32).at[0, :4].set(
        jnp.stack([bi0[0], bi0[1], bi1[0], bi1[1]]))

    sp2 = _run_phase_d(embA0, embR0, embA1, embR1, target_features,
                       interW, interb, bet4) / N
    ib = jax.nn.softmax(jnp.stack([jnp.dot(interatt[0], sp2[0]),
                                   jnp.dot(interatt[0], sp2[1]),
                                   jnp.dot(interatt[0], sp2[2])]))
    bet7 = bet4.at[0, 4:7].set(ib)

    return _run_phase_e(embA0, embR0, embA1, embR1, target_features, bet7)


# speedup vs baseline: 4.2344x; 1.2543x over previous
import functools

import jax
import jax.numpy as jnp
from jax import lax
from jax.experimental import pallas as pl
from jax.experimental.pallas import tpu as pltpu
from jax.experimental.pallas import tpu_sc as plsc

D = 128
N = 50000
M = 50000
K = 10
LANES = 16
BLK = 2000
GRID = N // BLK
NCHUNK = N // LANES
NWORK = 32
CPW = -(-NCHUNK // NWORK)
PADC = NWORK * CPW
PADN = PADC * LANES
PADI = PADC * K * LANES
NBUF = 3


def _elu(x):
    return jnp.where(x > 0, x, jnp.exp(x) - 1.0)


def _phase_a_body(nh0, r0, nh1, rl1, nh2, rl2,
                  Wr0, Wr20, Wr1, Wr21,
                  att0, att20, att1, att21,
                  br0, br20, br1, br21, out_n, out_m):
    cfgs = (
        (nh0, Wr0, br0, att0, nh1),
        (r0, Wr20, br20, att20, rl1),
        (nh0, Wr1, br1, att1, nh2),
        (r0, Wr21, br21, att21, rl2),
    )
    s1s, l1s, ps, qs = [], [], [], []
    for ref, W, bias, att, tab in cfgs:
        hr = jnp.dot(ref[...], W[...].T, preferred_element_type=jnp.float32)
        hr = hr + bias[...][None, :]
        a1 = att[0, :D]
        a2 = att[0, D:]
        s1s.append(jnp.dot(hr, a1, preferred_element_type=jnp.float32))
        l1s.append(jnp.sum(jnp.abs(hr), axis=1))
        t = tab[...]
        ps.append(jnp.dot(t, a2, preferred_element_type=jnp.float32))
        qs.append(jnp.sum(jnp.abs(t), axis=1))
    out_n[...] = jnp.stack(s1s + l1s, axis=1)
    out_m[...] = jnp.stack(ps + qs, axis=1)


def _run_phase_a(nh0, r0, nh1, rl1, nh2, rl2, Ws, atts, brs):
    blk = pl.BlockSpec((BLK, D), lambda i: (i, 0))
    full = lambda a: pl.BlockSpec(a.shape, lambda i: tuple(0 for _ in a.shape))
    ins = [nh0, r0, nh1, rl1, nh2, rl2] + list(Ws) + list(atts) + list(brs)
    in_specs = [blk] * 6 + [full(a) for a in ins[6:]]
    return pl.pallas_call(
        _phase_a_body,
        grid=(GRID,),
        in_specs=in_specs,
        out_specs=[pl.BlockSpec((BLK, 8), lambda i: (i, 0))] * 2,
        out_shape=[jax.ShapeDtypeStruct((N, 8), jnp.float32),
                   jax.ShapeDtypeStruct((M, 8), jnp.float32)],
    )(*ins)


def _gather_body(table, idxF, s1h, l1h, ph, qh, out,
                 p_v, q_v, idx_a, s1_a, l1_a, w_v, row_v, acc_v,
                 sa, sb, sc):
    sems = (sa, sb, sc)
    info = plsc.get_sparse_core_info()
    nc = info.num_cores
    wid = lax.axis_index("s") * nc + lax.axis_index("c")
    pltpu.sync_copy(ph, p_v)
    pltpu.sync_copy(qh, q_v)
    start = wid * CPW
    end = jnp.minimum(start + CPW, NCHUNK)
    pltpu.sync_copy(idxF.at[pl.ds(start * (K * LANES), CPW * K * LANES)],
                    idx_a)
    pltpu.sync_copy(s1h.at[pl.ds(start * LANES, CPW * LANES)], s1_a)
    pltpu.sync_copy(l1h.at[pl.ds(start * LANES, CPW * LANES)], l1_a)

    def chunk(c, carry):
        cl = c - start
        ibase = cl * (K * LANES)
        nbase = cl * LANES
        s1r = s1_a[pl.ds(nbase, LANES)]
        l1r = l1_a[pl.ds(nbase, LANES)]
        es = []
        for k in range(K):
            ii = idx_a[pl.ds(ibase + k * LANES, LANES)]
            pk = plsc.load_gather(p_v, [ii])
            qk = plsc.load_gather(q_v, [ii])
            e = (s1r + pk) / jnp.maximum(l1r + qk, 1e-12)
            es.append(jnp.where(e > 0, e, 0.01 * e))
        m = es[0]
        for k in range(1, K):
            m = jnp.maximum(m, es[k])
        exps = [jnp.exp(e - m) for e in es]
        ssum = exps[0]
        for k in range(1, K):
            ssum = ssum + exps[k]
        inv = 1.0 / ssum
        for k in range(K):
            w_v[k] = exps[k] * inv

        def fire(k):
            pltpu.make_async_copy(
                table.at[idx_a.at[pl.ds(ibase + k * LANES, LANES)]],
                row_v.at[k % NBUF], sems[k % NBUF]).start()

        for k in range(NBUF):
            fire(k)
        for k in range(K):
            buf = k % NBUF
            pltpu.make_async_copy(
                table.at[idx_a.at[pl.ds(ibase + k * LANES, LANES)]],
                row_v.at[buf], sems[buf]).wait()

            kf = jnp.full((LANES,), k, jnp.int32)
            for n in range(LANES):
                lane = jnp.full((LANES,), n, jnp.int32)
                w = plsc.load_gather(w_v, [kf, lane])
                for j in range(D // LANES):
                    sl = pl.ds(j * LANES, LANES)
                    r = row_v[buf, n, sl]
                    if k == 0:
                        acc_v[n, sl] = w * r
                    else:
                        acc_v[n, sl] = acc_v[n, sl] + w * r
            if k + NBUF < K:
                fire(k + NBUF)
        pltpu.sync_copy(acc_v, out.at[pl.ds(c * LANES, LANES), :])
        return carry

    lax.fori_loop(start, end, chunk, 0)


@functools.lru_cache(maxsize=None)
def _get_gather_call():
    mesh = plsc.VectorSubcoreMesh(core_axis_name="c", subcore_axis_name="s")
    return pl.kernel(
        _gather_body,
        out_type=jax.ShapeDtypeStruct((N, D), jnp.float32),
        mesh=mesh,
        compiler_params=pltpu.CompilerParams(needs_layout_passes=False),
        scratch_types=[
        pltpu.VMEM((M,), jnp.float32),
        pltpu.VMEM((M,), jnp.float32),
        pltpu.VMEM((CPW * K * LANES,), jnp.int32),
        pltpu.VMEM((CPW * LANES,), jnp.float32),
        pltpu.VMEM((CPW * LANES,), jnp.float32),
        pltpu.VMEM((K, LANES), jnp.float32),
        pltpu.VMEM((NBUF, LANES, D), jnp.float32),
        pltpu.VMEM((LANES, D), jnp.float32),
        pltpu.SemaphoreType.DMA,
        pltpu.SemaphoreType.DMA,
        pltpu.SemaphoreType.DMA,
        ],
    )


def _phase_c_body(a0, r0, a1, r1, w0, b0, w1, b1, out):
    i = pl.program_id(0)
    rows = []
    for em, W, bb in ((a0, w0, b0), (r0, w0, b0), (a1, w1, b1), (r1, w1, b1)):
        t = jnp.tanh(jnp.dot(em[...], W[...].T,
                             preferred_element_type=jnp.float32)
                     + bb[...][None, :])
        rows.append(jnp.sum(t, axis=0))
    val = jnp.stack(rows, axis=0)

    @pl.when(i == 0)
    def _():
        out[...] = val

    @pl.when(i > 0)
    def _():
        out[...] = out[...] + val


def _run_phase_c(a0, r0, a1, r1, w0, b0, w1, b1):
    blk = pl.BlockSpec((BLK, D), lambda i: (i, 0))
    full = lambda a: pl.BlockSpec(a.shape, lambda i: tuple(0 for _ in a.shape))
    return pl.pallas_call(
        _phase_c_body,
        grid=(GRID,),
        in_specs=[blk] * 4 + [full(w0), full(b0), full(w1), full(b1)],
        out_specs=pl.BlockSpec((4, D), lambda i: (0, 0)),
        out_shape=jax.ShapeDtypeStruct((4, D), jnp.float32),
    )(a0, r0, a1, r1, w0, b0, w1, b1)


def _phase_d_body(a0, r0, a1, r1, tf, W, bb, bet, out):
    i = pl.program_id(0)
    ba0 = bet[0, 0]
    br0 = bet[0, 1]
    ba1 = bet[0, 2]
    br1 = bet[0, 3]
    e0 = _elu(ba0 * a0[...] + br0 * r0[...])
    e1 = _elu(ba1 * a1[...] + br1 * r1[...])
    rows = []
    for em in (e0, e1, tf[...]):
        t = jnp.tanh(jnp.dot(em, W[...].T,
                             preferred_element_type=jnp.float32)
                     + bb[...][None, :])
        rows.append(jnp.sum(t, axis=0))
    rows.append(jnp.zeros((D,), jnp.float32))
    val = jnp.stack(rows, axis=0)

    @pl.when(i == 0)
    def _():
        out[...] = val

    @pl.when(i > 0)
    def _():
        out[...] = out[...] + val


def _run_phase_d(a0, r0, a1, r1, tf, W, bb, bet):
    blk = pl.BlockSpec((BLK, D), lambda i: (i, 0))
    full = lambda a: pl.BlockSpec(a.shape, lambda i: tuple(0 for _ in a.shape))
    return pl.pallas_call(
        _phase_d_body,
        grid=(GRID,),
        in_specs=[blk] * 5 + [full(W), full(bb), full(bet)],
        out_specs=pl.BlockSpec((4, D), lambda i: (0, 0)),
        out_shape=jax.ShapeDtypeStruct((4, D), jnp.float32),
    )(a0, r0, a1, r1, tf, W, bb, bet)


def _phase_e_body(a0, r0, a1, r1, tf, bet, out):
    ba0 = bet[0, 0]
    br0 = bet[0, 1]
    ba1 = bet[0, 2]
    br1 = bet[0, 3]
    ib0 = bet[0, 4]
    ib1 = bet[0, 5]
    ib2 = bet[0, 6]
    e0 = _elu(ba0 * a0[...] + br0 * r0[...])
    e1 = _elu(ba1 * a1[...] + br1 * r1[...])
    out[...] = ib0 * e0 + ib1 * e1 + ib2 * tf[...]


def _run_phase_e(a0, r0, a1, r1, tf, bet):
    blk = pl.BlockSpec((BLK, D), lambda i: (i, 0))
    full = lambda a: pl.BlockSpec(a.shape, lambda i: tuple(0 for _ in a.shape))
    return pl.pallas_call(
        _phase_e_body,
        grid=(GRID,),
        in_specs=[blk] * 5 + [full(bet)],
        out_specs=blk,
        out_shape=jax.ShapeDtypeStruct((N, D), jnp.float32),
    )(a0, r0, a1, r1, tf, bet)


def kernel(nei_h_0, nei_h_1, nei_h_2, rel_0, rel_1, rel_2, target_features,
           att0, att2_0, Wr0, br0, Wr2_0, br2_0, aggW0, aggb0, aggatt0,
           att1, att2_1, Wr1, br1, Wr2_1, br2_1, aggW1, aggb1, aggatt1,
           interW, interb, interatt, sele_nei_0, sele_nei_1):
    f32 = jnp.float32

    def _chunk_kmajor(nei):
        r = nei.reshape(NCHUNK, LANES, K).astype(jnp.int32)
        flat = jnp.ravel(jnp.transpose(r, (0, 2, 1)))
        return jnp.pad(flat, (0, PADI - flat.shape[0]))

    idxF0 = _chunk_kmajor(sele_nei_0)
    idxF1 = _chunk_kmajor(sele_nei_1)

    scal_n, scal_m = _run_phase_a(
        nei_h_0, rel_0, nei_h_1, rel_1, nei_h_2, rel_2,
        (Wr0, Wr2_0, Wr1, Wr2_1),
        (att0, att2_0, att1, att2_1),
        (br0, br2_0, br1, br2_1))
    sn = jnp.pad(jnp.transpose(scal_n),
                 ((0, 0), (0, PADN - N)))
    sm = jnp.transpose(scal_m)

    tables = (nei_h_1, rel_1, nei_h_2, rel_2)
    idxTs = (idxF0, idxF0, idxF1, idxF1)
    gather_call = _get_gather_call()
    embs = []
    for b in range(4):
        embs.append(gather_call(
            tables[b], idxTs[b], sn[b], sn[4 + b], sm[b], sm[4 + b]))
    embA0, embR0, embA1, embR1 = embs

    sp = _run_phase_c(embA0, embR0, embA1, embR1,
                      aggW0, aggb0, aggW1, aggb1) / N
    bi0 = jax.nn.softmax(jnp.stack([jnp.dot(aggatt0[0], sp[0]),
                                    jnp.dot(aggatt0[0], sp[1])]))
    bi1 = jax.nn.softmax(jnp.stack([jnp.dot(aggatt1[0], sp[2]),
                                    jnp.dot(aggatt1[0], sp[3])]))
    bet4 = jnp.zeros((1, D), f32).at[0, :4].set(
        jnp.stack([bi0[0], bi0[1], bi1[0], bi1[1]]))

    sp2 = _run_phase_d(embA0, embR0, embA1, embR1, target_features,
                       interW, interb, bet4) / N
    ib = jax.nn.softmax(jnp.stack([jnp.dot(interatt[0], sp2[0]),
                                   jnp.dot(interatt[0], sp2[1]),
                                   jnp.dot(interatt[0], sp2[2])]))
    bet7 = bet4.at[0, 4:7].set(ib)

    return _run_phase_e(embA0, embR0, embA1, embR1, target_features, bet7)


# speedup vs baseline: 5.3076x; 1.2535x over previous
import functools

import jax
import jax.numpy as jnp
from jax import lax
from jax.experimental import pallas as pl
from jax.experimental.pallas import tpu as pltpu
from jax.experimental.pallas import tpu_sc as plsc

D = 128
N = 50000
M = 50000
K = 10
LANES = 16
CKL = K * LANES
BLK = 2000
GRID = N // BLK
NCHUNK = N // LANES
NWORK = 32
CPW = -(-NCHUNK // NWORK)
PADC = NWORK * CPW
PADI = PADC * CKL
EROWS = N * K
CB = 25
RGRID = NCHUNK // CB


def _elu(x):
    return jnp.where(x > 0, x, jnp.exp(x) - 1.0)


def _phase_a_body(nh0, r0, Wr0, Wr20, Wr1, Wr21,
                  att0, att20, att1, att21,
                  br0, br20, br1, br21, out_n):
    cfgs = (
        (nh0, Wr0, br0, att0),
        (r0, Wr20, br20, att20),
        (nh0, Wr1, br1, att1),
        (r0, Wr21, br21, att21),
    )
    s1s, l1s = [], []
    for ref, W, bias, att in cfgs:
        hr = jnp.dot(ref[...], W[...].T, preferred_element_type=jnp.float32)
        hr = hr + bias[...][None, :]
        s1s.append(jnp.dot(hr, att[0, :D], preferred_element_type=jnp.float32))
        l1s.append(jnp.sum(jnp.abs(hr), axis=1))
    out_n[...] = jnp.stack(s1s + l1s, axis=1)


def _run_phase_a(nh0, r0, Ws, atts, brs):
    blk = pl.BlockSpec((BLK, D), lambda i: (i, 0))
    full = lambda a: pl.BlockSpec(a.shape, lambda i: tuple(0 for _ in a.shape))
    ins = [nh0, r0] + list(Ws) + list(atts) + list(brs)
    in_specs = [blk] * 2 + [full(a) for a in ins[2:]]
    return pl.pallas_call(
        _phase_a_body,
        grid=(GRID,),
        in_specs=in_specs,
        out_specs=pl.BlockSpec((BLK, 8), lambda i: (i, 0)),
        out_shape=jax.ShapeDtypeStruct((N, 8), jnp.float32),
    )(*ins)


def _pump_body(t0, t1, t2, t3, idx0, idx1, e0, e1, e2, e3,
               i0a, i1a, r0, r1, r2, r3,
               g0, g1, g2, g3, o0, o1, o2, o3):
    info = plsc.get_sparse_core_info()
    wid = lax.axis_index("s") * info.num_cores + lax.axis_index("c")
    start = wid * CPW
    end = jnp.minimum(start + CPW, NCHUNK)
    pltpu.sync_copy(idx0.at[pl.ds(start * CKL, CPW * CKL)], i0a)
    pltpu.sync_copy(idx1.at[pl.ds(start * CKL, CPW * CKL)], i1a)
    jobs = ((t0, i0a, e0, r0, g0, o0),
            (t1, i0a, e1, r1, g1, o1),
            (t2, i1a, e2, r2, g2, o2),
            (t3, i1a, e3, r3, g3, o3))

    def g_copy(c, job):
        t, ia, e, r, g, o = job
        cl = c - start
        return pltpu.make_async_copy(
            t.at[ia.at[pl.ds(cl * CKL, CKL)]], r, g)

    def o_copy(c, job):
        t, ia, e, r, g, o = job
        return pltpu.make_async_copy(r, e.at[pl.ds(c * CKL, CKL), :], o)

    for job in jobs:
        g_copy(start, job).start()

    def chunk(c, carry):
        for job in jobs:
            g_copy(c, job).wait()
            o_copy(c, job).start()
        cn = jnp.minimum(c + 1, end - 1)
        for job in jobs:
            o_copy(c, job).wait()
            g_copy(cn, job).start()
        return carry

    lax.fori_loop(start, end, chunk, 0)
    for job in jobs:
        g_copy(end - 1, job).wait()


@functools.lru_cache(maxsize=None)
def _get_pump_call():
    mesh = plsc.VectorSubcoreMesh(core_axis_name="c", subcore_axis_name="s")
    return pl.kernel(
        _pump_body,
        out_type=[jax.ShapeDtypeStruct((EROWS, D), jnp.float32)] * 4,
        mesh=mesh,
        compiler_params=pltpu.CompilerParams(needs_layout_passes=False),
        scratch_types=[
            pltpu.VMEM((CPW * CKL,), jnp.int32),
            pltpu.VMEM((CPW * CKL,), jnp.int32),
            pltpu.VMEM((CKL, D), jnp.float32),
            pltpu.VMEM((CKL, D), jnp.float32),
            pltpu.VMEM((CKL, D), jnp.float32),
            pltpu.VMEM((CKL, D), jnp.float32),
            pltpu.SemaphoreType.DMA,
            pltpu.SemaphoreType.DMA,
            pltpu.SemaphoreType.DMA,
            pltpu.SemaphoreType.DMA,
            pltpu.SemaphoreType.DMA,
            pltpu.SemaphoreType.DMA,
            pltpu.SemaphoreType.DMA,
            pltpu.SemaphoreType.DMA,
        ],
    )


def _reduce_body(e0, e1, e2, e3, sn, a0, a1, a2, a3,
                 out0, out1, out2, out3):
    outs = (out0, out1, out2, out3)
    for b, (eref, aref) in enumerate(((e0, a0), (e1, a1), (e2, a2),
                                      (e3, a3))):
        rows = eref[...]
        p = jnp.dot(rows, aref[0, D:],
                    preferred_element_type=jnp.float32)
        q = jnp.sum(jnp.abs(rows), axis=1)
        s1 = sn[:, b].reshape(CB, 1, LANES)
        l1 = sn[:, 4 + b].reshape(CB, 1, LANES)
        p4 = p.reshape(CB, K, LANES)
        q4 = q.reshape(CB, K, LANES)
        e = (s1 + p4) / jnp.maximum(l1 + q4, 1e-12)
        e = jnp.where(e > 0, e, 0.01 * e)
        m = jnp.max(e, axis=1, keepdims=True)
        ex = jnp.exp(e - m)
        w = ex / jnp.sum(ex, axis=1, keepdims=True)
        r4 = rows.reshape(CB, K, LANES, D)
        outs[b][...] = jnp.sum(w[..., None] * r4, axis=1).reshape(
            CB * LANES, D)


def _run_reduce(embs, scal_n, atts):
    eblk = pl.BlockSpec((CB * CKL, D), lambda i: (i, 0))
    nblk = pl.BlockSpec((CB * LANES, 8), lambda i: (i, 0))
    full = lambda a: pl.BlockSpec(a.shape, lambda i: tuple(0 for _ in a.shape))
    oblk = pl.BlockSpec((CB * LANES, D), lambda i: (i, 0))
    return pl.pallas_call(
        _reduce_body,
        grid=(RGRID,),
        in_specs=[eblk] * 4 + [nblk] + [full(a) for a in atts],
        out_specs=[oblk] * 4,
        out_shape=[jax.ShapeDtypeStruct((N, D), jnp.float32)] * 4,
    )(*embs, scal_n, *atts)


def _phase_c_body(a0, r0, a1, r1, w0, b0, w1, b1, out):
    i = pl.program_id(0)
    rows = []
    for em, W, bb in ((a0, w0, b0), (r0, w0, b0), (a1, w1, b1), (r1, w1, b1)):
        t = jnp.tanh(jnp.dot(em[...], W[...].T,
                             preferred_element_type=jnp.float32)
                     + bb[...][None, :])
        rows.append(jnp.sum(t, axis=0))
    val = jnp.stack(rows, axis=0)

    @pl.when(i == 0)
    def _():
        out[...] = val

    @pl.when(i > 0)
    def _():
        out[...] = out[...] + val


def _run_phase_c(a0, r0, a1, r1, w0, b0, w1, b1):
    blk = pl.BlockSpec((BLK, D), lambda i: (i, 0))
    full = lambda a: pl.BlockSpec(a.shape, lambda i: tuple(0 for _ in a.shape))
    return pl.pallas_call(
        _phase_c_body,
        grid=(GRID,),
        in_specs=[blk] * 4 + [full(w0), full(b0), full(w1), full(b1)],
        out_specs=pl.BlockSpec((4, D), lambda i: (0, 0)),
        out_shape=jax.ShapeDtypeStruct((4, D), jnp.float32),
    )(a0, r0, a1, r1, w0, b0, w1, b1)


def _phase_d_body(a0, r0, a1, r1, tf, W, bb, bet, out):
    i = pl.program_id(0)
    ba0 = bet[0, 0]
    br0 = bet[0, 1]
    ba1 = bet[0, 2]
    br1 = bet[0, 3]
    e0 = _elu(ba0 * a0[...] + br0 * r0[...])
    e1 = _elu(ba1 * a1[...] + br1 * r1[...])
    rows = []
    for em in (e0, e1, tf[...]):
        t = jnp.tanh(jnp.dot(em, W[...].T,
                             preferred_element_type=jnp.float32)
                     + bb[...][None, :])
        rows.append(jnp.sum(t, axis=0))
    rows.append(jnp.zeros((D,), jnp.float32))
    val = jnp.stack(rows, axis=0)

    @pl.when(i == 0)
    def _():
        out[...] = val

    @pl.when(i > 0)
    def _():
        out[...] = out[...] + val


def _run_phase_d(a0, r0, a1, r1, tf, W, bb, bet):
    blk = pl.BlockSpec((BLK, D), lambda i: (i, 0))
    full = lambda a: pl.BlockSpec(a.shape, lambda i: tuple(0 for _ in a.shape))
    return pl.pallas_call(
        _phase_d_body,
        grid=(GRID,),
        in_specs=[blk] * 5 + [full(W), full(bb), full(bet)],
        out_specs=pl.BlockSpec((4, D), lambda i: (0, 0)),
        out_shape=jax.ShapeDtypeStruct((4, D), jnp.float32),
    )(a0, r0, a1, r1, tf, W, bb, bet)


def _phase_e_body(a0, r0, a1, r1, tf, bet, out):
    ba0 = bet[0, 0]
    br0 = bet[0, 1]
    ba1 = bet[0, 2]
    br1 = bet[0, 3]
    ib0 = bet[0, 4]
    ib1 = bet[0, 5]
    ib2 = bet[0, 6]
    e0 = _elu(ba0 * a0[...] + br0 * r0[...])
    e1 = _elu(ba1 * a1[...] + br1 * r1[...])
    out[...] = ib0 * e0 + ib1 * e1 + ib2 * tf[...]


def _run_phase_e(a0, r0, a1, r1, tf, bet):
    blk = pl.BlockSpec((BLK, D), lambda i: (i, 0))
    full = lambda a: pl.BlockSpec(a.shape, lambda i: tuple(0 for _ in a.shape))
    return pl.pallas_call(
        _phase_e_body,
        grid=(GRID,),
        in_specs=[blk] * 5 + [full(bet)],
        out_specs=blk,
        out_shape=jax.ShapeDtypeStruct((N, D), jnp.float32),
    )(a0, r0, a1, r1, tf, bet)


def kernel(nei_h_0, nei_h_1, nei_h_2, rel_0, rel_1, rel_2, target_features,
           att0, att2_0, Wr0, br0, Wr2_0, br2_0, aggW0, aggb0, aggatt0,
           att1, att2_1, Wr1, br1, Wr2_1, br2_1, aggW1, aggb1, aggatt1,
           interW, interb, interatt, sele_nei_0, sele_nei_1):
    f32 = jnp.float32

    def _chunk_kmajor(nei):
        r = nei.reshape(NCHUNK, LANES, K).astype(jnp.int32)
        flat = jnp.ravel(jnp.transpose(r, (0, 2, 1)))
        return jnp.pad(flat, (0, PADI - flat.shape[0]))

    idxF0 = _chunk_kmajor(sele_nei_0)
    idxF1 = _chunk_kmajor(sele_nei_1)

    scal_n = _run_phase_a(
        nei_h_0, rel_0,
        (Wr0, Wr2_0, Wr1, Wr2_1),
        (att0, att2_0, att1, att2_1),
        (br0, br2_0, br1, br2_1))

    dense = _get_pump_call()(
        nei_h_1, rel_1, nei_h_2, rel_2, idxF0, idxF1)
    embA0, embR0, embA1, embR1 = _run_reduce(
        dense, scal_n, (att0, att2_0, att1, att2_1))

    sp = _run_phase_c(embA0, embR0, embA1, embR1,
                      aggW0, aggb0, aggW1, aggb1) / N
    bi0 = jax.nn.softmax(jnp.stack([jnp.dot(aggatt0[0], sp[0]),
                                    jnp.dot(aggatt0[0], sp[1])]))
    bi1 = jax.nn.softmax(jnp.stack([jnp.dot(aggatt1[0], sp[2]),
                                    jnp.dot(aggatt1[0], sp[3])]))
    bet4 = jnp.zeros((1, D), f32).at[0, :4].set(
        jnp.stack([bi0[0], bi0[1], bi1[0], bi1[1]]))

    sp2 = _run_phase_d(embA0, embR0, embA1, embR1, target_features,
                       interW, interb, bet4) / N
    ib = jax.nn.softmax(jnp.stack([jnp.dot(interatt[0], sp2[0]),
                                   jnp.dot(interatt[0], sp2[1]),
                                   jnp.dot(interatt[0], sp2[2])]))
    bet7 = bet4.at[0, 4:7].set(ib)

    return _run_phase_e(embA0, embR0, embA1, embR1, target_features, bet7)


# speedup vs baseline: 6.4246x; 1.2104x over previous
import functools

import jax
import jax.numpy as jnp
from jax import lax
from jax.experimental import pallas as pl
from jax.experimental.pallas import tpu as pltpu
from jax.experimental.pallas import tpu_sc as plsc

D = 128
N = 50000
M = 50000
K = 10
LANES = 16
CKL = K * LANES
BLK = 2000
GRID = N // BLK
NCHUNK = N // LANES
NWORK = 32
SEG = 5
SN = N // SEG
SCHUNK = SN // LANES
CPW = -(-SCHUNK // NWORK)
PADC = NWORK * CPW
PADI = PADC * CKL
SROWS = SN * K
CB = 25
RGRID = SCHUNK // CB


def _elu(x):
    return jnp.where(x > 0, x, jnp.exp(x) - 1.0)


def _phase_a_body(nh0, r0, Wr0, Wr20, Wr1, Wr21,
                  att0, att20, att1, att21,
                  br0, br20, br1, br21, out_n):
    cfgs = (
        (nh0, Wr0, br0, att0),
        (r0, Wr20, br20, att20),
        (nh0, Wr1, br1, att1),
        (r0, Wr21, br21, att21),
    )
    s1s, l1s = [], []
    for ref, W, bias, att in cfgs:
        hr = jnp.dot(ref[...], W[...].T, preferred_element_type=jnp.float32)
        hr = hr + bias[...][None, :]
        s1s.append(jnp.dot(hr, att[0, :D], preferred_element_type=jnp.float32))
        l1s.append(jnp.sum(jnp.abs(hr), axis=1))
    out_n[...] = jnp.stack(s1s + l1s, axis=1)


def _run_phase_a(nh0, r0, Ws, atts, brs):
    blk = pl.BlockSpec((BLK, D), lambda i: (i, 0))
    full = lambda a: pl.BlockSpec(a.shape, lambda i: tuple(0 for _ in a.shape))
    ins = [nh0, r0] + list(Ws) + list(atts) + list(brs)
    in_specs = [blk] * 2 + [full(a) for a in ins[2:]]
    return pl.pallas_call(
        _phase_a_body,
        grid=(GRID,),
        in_specs=in_specs,
        out_specs=pl.BlockSpec((BLK, 8), lambda i: (i, 0)),
        out_shape=jax.ShapeDtypeStruct((N, 8), jnp.float32),
    )(*ins)


def _pump_body(t0, t1, t2, t3, idx0, idx1, e0, e1, e2, e3,
               i0a, i1a, r0, r1, r2, r3,
               g0, g1, g2, g3, o0, o1, o2, o3):
    info = plsc.get_sparse_core_info()
    wid = lax.axis_index("s") * info.num_cores + lax.axis_index("c")
    start = wid * CPW
    end = jnp.minimum(start + CPW, SCHUNK)
    pltpu.sync_copy(idx0.at[pl.ds(start * CKL, CPW * CKL)], i0a)
    pltpu.sync_copy(idx1.at[pl.ds(start * CKL, CPW * CKL)], i1a)
    jobs = ((t0, i0a, e0, r0, g0, o0),
            (t1, i0a, e1, r1, g1, o1),
            (t2, i1a, e2, r2, g2, o2),
            (t3, i1a, e3, r3, g3, o3))

    def g_copy(c, job):
        t, ia, e, r, g, o = job
        cl = c - start
        return pltpu.make_async_copy(
            t.at[ia.at[pl.ds(cl * CKL, CKL)]], r, g)

    def o_copy(c, job):
        t, ia, e, r, g, o = job
        return pltpu.make_async_copy(r, e.at[pl.ds(c * CKL, CKL), :], o)

    for job in jobs:
        g_copy(start, job).start()

    def chunk(c, carry):
        for job in jobs:
            g_copy(c, job).wait()
            o_copy(c, job).start()
        cn = jnp.minimum(c + 1, end - 1)
        for job in jobs:
            o_copy(c, job).wait()
            g_copy(cn, job).start()
        return carry

    lax.fori_loop(start, end, chunk, 0)
    for job in jobs:
        g_copy(end - 1, job).wait()


@functools.lru_cache(maxsize=None)
def _get_pump_call():
    mesh = plsc.VectorSubcoreMesh(core_axis_name="c", subcore_axis_name="s")
    return pl.kernel(
        _pump_body,
        out_type=[jax.ShapeDtypeStruct((SROWS, D), jnp.float32)] * 4,
        mesh=mesh,
        compiler_params=pltpu.CompilerParams(needs_layout_passes=False),
        scratch_types=[
            pltpu.VMEM((CPW * CKL,), jnp.int32),
            pltpu.VMEM((CPW * CKL,), jnp.int32),
            pltpu.VMEM((CKL, D), jnp.float32),
            pltpu.VMEM((CKL, D), jnp.float32),
            pltpu.VMEM((CKL, D), jnp.float32),
            pltpu.VMEM((CKL, D), jnp.float32),
            pltpu.SemaphoreType.DMA,
            pltpu.SemaphoreType.DMA,
            pltpu.SemaphoreType.DMA,
            pltpu.SemaphoreType.DMA,
            pltpu.SemaphoreType.DMA,
            pltpu.SemaphoreType.DMA,
            pltpu.SemaphoreType.DMA,
            pltpu.SemaphoreType.DMA,
        ],
    )


def _reduce_body(e0, e1, e2, e3, sn, a0, a1, a2, a3,
                 out0, out1, out2, out3):
    outs = (out0, out1, out2, out3)
    for b, (eref, aref) in enumerate(((e0, a0), (e1, a1), (e2, a2),
                                      (e3, a3))):
        rows = eref[...]
        p = jnp.dot(rows, aref[0, D:],
                    preferred_element_type=jnp.float32)
        q = jnp.sum(jnp.abs(rows), axis=1)
        s1 = sn[:, b].reshape(CB, 1, LANES)
        l1 = sn[:, 4 + b].reshape(CB, 1, LANES)
        p4 = p.reshape(CB, K, LANES)
        q4 = q.reshape(CB, K, LANES)
        e = (s1 + p4) / jnp.maximum(l1 + q4, 1e-12)
        e = jnp.where(e > 0, e, 0.01 * e)
        m = jnp.max(e, axis=1, keepdims=True)
        ex = jnp.exp(e - m)
        w = ex / jnp.sum(ex, axis=1, keepdims=True)
        r4 = rows.reshape(CB, K, LANES, D)
        outs[b][...] = jnp.sum(w[..., None] * r4, axis=1).reshape(
            CB * LANES, D)


def _run_reduce(embs, scal_n, atts):
    eblk = pl.BlockSpec((CB * CKL, D), lambda i: (i, 0))
    nblk = pl.BlockSpec((CB * LANES, 8), lambda i: (i, 0))
    full = lambda a: pl.BlockSpec(a.shape, lambda i: tuple(0 for _ in a.shape))
    oblk = pl.BlockSpec((CB * LANES, D), lambda i: (i, 0))
    return pl.pallas_call(
        _reduce_body,
        grid=(RGRID,),
        in_specs=[eblk] * 4 + [nblk] + [full(a) for a in atts],
        out_specs=[oblk] * 4,
        out_shape=[jax.ShapeDtypeStruct((SN, D), jnp.float32)] * 4,
    )(*embs, scal_n, *atts)


def _phase_c_body(a0, r0, a1, r1, w0, b0, w1, b1, out):
    i = pl.program_id(0)
    rows = []
    for em, W, bb in ((a0, w0, b0), (r0, w0, b0), (a1, w1, b1), (r1, w1, b1)):
        t = jnp.tanh(jnp.dot(em[...], W[...].T,
                             preferred_element_type=jnp.float32)
                     + bb[...][None, :])
        rows.append(jnp.sum(t, axis=0))
    val = jnp.stack(rows, axis=0)

    @pl.when(i == 0)
    def _():
        out[...] = val

    @pl.when(i > 0)
    def _():
        out[...] = out[...] + val


def _run_phase_c(a0, r0, a1, r1, w0, b0, w1, b1):
    blk = pl.BlockSpec((BLK, D), lambda i: (i, 0))
    full = lambda a: pl.BlockSpec(a.shape, lambda i: tuple(0 for _ in a.shape))
    return pl.pallas_call(
        _phase_c_body,
        grid=(GRID,),
        in_specs=[blk] * 4 + [full(w0), full(b0), full(w1), full(b1)],
        out_specs=pl.BlockSpec((4, D), lambda i: (0, 0)),
        out_shape=jax.ShapeDtypeStruct((4, D), jnp.float32),
    )(a0, r0, a1, r1, w0, b0, w1, b1)


def _phase_d_body(a0, r0, a1, r1, tf, W, bb, bet, out):
    i = pl.program_id(0)
    ba0 = bet[0, 0]
    br0 = bet[0, 1]
    ba1 = bet[0, 2]
    br1 = bet[0, 3]
    e0 = _elu(ba0 * a0[...] + br0 * r0[...])
    e1 = _elu(ba1 * a1[...] + br1 * r1[...])
    rows = []
    for em in (e0, e1, tf[...]):
        t = jnp.tanh(jnp.dot(em, W[...].T,
                             preferred_element_type=jnp.float32)
                     + bb[...][None, :])
        rows.append(jnp.sum(t, axis=0))
    rows.append(jnp.zeros((D,), jnp.float32))
    val = jnp.stack(rows, axis=0)

    @pl.when(i == 0)
    def _():
        out[...] = val

    @pl.when(i > 0)
    def _():
        out[...] = out[...] + val


def _run_phase_d(a0, r0, a1, r1, tf, W, bb, bet):
    blk = pl.BlockSpec((BLK, D), lambda i: (i, 0))
    full = lambda a: pl.BlockSpec(a.shape, lambda i: tuple(0 for _ in a.shape))
    return pl.pallas_call(
        _phase_d_body,
        grid=(GRID,),
        in_specs=[blk] * 5 + [full(W), full(bb), full(bet)],
        out_specs=pl.BlockSpec((4, D), lambda i: (0, 0)),
        out_shape=jax.ShapeDtypeStruct((4, D), jnp.float32),
    )(a0, r0, a1, r1, tf, W, bb, bet)


def _phase_e_body(a0, r0, a1, r1, tf, bet, out):
    ba0 = bet[0, 0]
    br0 = bet[0, 1]
    ba1 = bet[0, 2]
    br1 = bet[0, 3]
    ib0 = bet[0, 4]
    ib1 = bet[0, 5]
    ib2 = bet[0, 6]
    e0 = _elu(ba0 * a0[...] + br0 * r0[...])
    e1 = _elu(ba1 * a1[...] + br1 * r1[...])
    out[...] = ib0 * e0 + ib1 * e1 + ib2 * tf[...]


def _run_phase_e(a0, r0, a1, r1, tf, bet):
    blk = pl.BlockSpec((BLK, D), lambda i: (i, 0))
    full = lambda a: pl.BlockSpec(a.shape, lambda i: tuple(0 for _ in a.shape))
    return pl.pallas_call(
        _phase_e_body,
        grid=(GRID,),
        in_specs=[blk] * 5 + [full(bet)],
        out_specs=blk,
        out_shape=jax.ShapeDtypeStruct((N, D), jnp.float32),
    )(a0, r0, a1, r1, tf, bet)


def kernel(nei_h_0, nei_h_1, nei_h_2, rel_0, rel_1, rel_2, target_features,
           att0, att2_0, Wr0, br0, Wr2_0, br2_0, aggW0, aggb0, aggatt0,
           att1, att2_1, Wr1, br1, Wr2_1, br2_1, aggW1, aggb1, aggatt1,
           interW, interb, interatt, sele_nei_0, sele_nei_1):
    f32 = jnp.float32

    def _chunk_kmajor(nei):
        r = nei.reshape(SCHUNK, LANES, K).astype(jnp.int32)
        flat = jnp.ravel(jnp.transpose(r, (0, 2, 1)))
        return jnp.pad(flat, (0, PADI - flat.shape[0]))

    scal_n = _run_phase_a(
        nei_h_0, rel_0,
        (Wr0, Wr2_0, Wr1, Wr2_1),
        (att0, att2_0, att1, att2_1),
        (br0, br2_0, br1, br2_1))

    pump = _get_pump_call()
    segs = []
    for s in range(SEG):
        idxF0 = _chunk_kmajor(lax.slice(sele_nei_0, (s * SN, 0),
                                        ((s + 1) * SN, K)))
        idxF1 = _chunk_kmajor(lax.slice(sele_nei_1, (s * SN, 0),
                                        ((s + 1) * SN, K)))
        dense = pump(nei_h_1, rel_1, nei_h_2, rel_2, idxF0, idxF1)
        segs.append(_run_reduce(
            dense, lax.slice(scal_n, (s * SN, 0), ((s + 1) * SN, 8)),
            (att0, att2_0, att1, att2_1)))
    embA0, embR0, embA1, embR1 = (
        jnp.concatenate([sg[b] for sg in segs], axis=0) for b in range(4))

    sp = _run_phase_c(embA0, embR0, embA1, embR1,
                      aggW0, aggb0, aggW1, aggb1) / N
    bi0 = jax.nn.softmax(jnp.stack([jnp.dot(aggatt0[0], sp[0]),
                                    jnp.dot(aggatt0[0], sp[1])]))
    bi1 = jax.nn.softmax(jnp.stack([jnp.dot(aggatt1[0], sp[2]),
                                    jnp.dot(aggatt1[0], sp[3])]))
    bet4 = jnp.zeros((1, D), f32).at[0, :4].set(
        jnp.stack([bi0[0], bi0[1], bi1[0], bi1[1]]))

    sp2 = _run_phase_d(embA0, embR0, embA1, embR1, target_features,
                       interW, interb, bet4) / N
    ib = jax.nn.softmax(jnp.stack([jnp.dot(interatt[0], sp2[0]),
                                   jnp.dot(interatt[0], sp2[1]),
                                   jnp.dot(interatt[0], sp2[2])]))
    bet7 = bet4.at[0, 4:7].set(ib)

    return _run_phase_e(embA0, embR0, embA1, embR1, target_features, bet7)
